# baseline (speedup 1.0000x reference)
import functools

import jax
import jax.numpy as jnp
from jax import lax
from jax.experimental import pallas as pl
from jax.experimental.pallas import tpu as pltpu
from jax.experimental.pallas import tpu_sc as plsc

ENT_DIM = 32
REL_DIM = 32
B = 16384
L = 50
NREL = 1000

_info = plsc.get_sparse_core_info()
NC, NS = _info.num_cores, _info.num_subcores
NW = NC * NS
BW = B // NW
BTW = BW // 128
NCHUNK = L * BTW
NBUF = 4


def _body(ent_hbm, rel_hbm, eidx_hbm, ridx_hbm, o5, o4,
          idx_v, idxc_v, gb0, gb1, gb2, gb3, tb0, tb1, rtab_v, ridx_v, rtile,
          sg0, sg1, sg2, sg3, sw0, sw1, srt):
    wid = lax.axis_index("s") * NC + lax.axis_index("c")
    b0 = wid * BW
    gbufs = (gb0, gb1, gb2, gb3)
    sgs = (sg0, sg1, sg2, sg3)
    tbufs = (tb0, tb1)
    sws = (sw0, sw1)
    iota = lax.iota(jnp.int32, 16)

    pltpu.async_copy(rel_hbm, rtab_v, srt)
    pltpu.sync_copy(eidx_hbm.at[pl.ds(b0, BW)], idx_v)
    pltpu.sync_copy(ridx_hbm.at[pl.ds(b0, BW)], ridx_v)

    def repack_and_fire(c, s):
        k = c // L
        l = c % L
        lvec = jnp.full((16,), 0, jnp.int32) + l
        for t in range(8):
            rows = plsc.load_gather(idx_v, [k * 128 + 16 * t + iota, lvec])
            idxc_v[s, pl.ds(16 * t, 16)] = rows
        pltpu.async_copy(ent_hbm.at[idxc_v.at[s]], gbufs[s], sgs[s])

    def wait_writes(p):
        for jt in range(4):
            pltpu.make_async_copy(tbufs[p].at[pl.ds(8 * jt, 8)],
                                  o5.at[0, jt, 0], sws[p]).wait()

    def process(c, s, p, skip_wait=False):
        k = c // L
        l = c % L
        pltpu.make_async_copy(ent_hbm.at[idxc_v.at[s]], gbufs[s],
                              sgs[s]).wait()
        if not skip_wait:
            wait_writes(p)
        for j in range(ENT_DIM):
            jvec = jnp.full((16,), j, jnp.int32)
            for t in range(8):
                v = plsc.load_gather(gbufs[s], [16 * t + iota, jvec])
                tbufs[p][j, pl.ds(16 * t, 16)] = v
        for jt in range(4):
            pltpu.async_copy(tbufs[p].at[pl.ds(8 * jt, 8)],
                             o5.at[l, jt, wid * BTW + k], sws[p])

    for c in range(NBUF - 1):
        repack_and_fire(c, c)
    for s in range(NBUF):
        process(s, s, s % 2, skip_wait=(s < 2))
        repack_and_fire(s + NBUF - 1, (s + NBUF - 1) % NBUF)

    def step(kk, _):
        for s in range(NBUF):
            c = kk * NBUF + s
            process(c, s, s % 2)
            cf = c + NBUF - 1

            @pl.when(cf < NCHUNK)
            def _():
                repack_and_fire(cf, (s + NBUF - 1) % NBUF)
        return 0

    lax.fori_loop(1, NCHUNK // NBUF, step, 0)
    wait_writes(0)
    wait_writes(1)

    pltpu.make_async_copy(rel_hbm, rtab_v, srt).wait()
    for k in range(BTW):
        for jt in range(4):

            def rel_row(jr, _, _k=k, _jt=jt):
                jvec = jnp.full((16,), 0, jnp.int32) + (8 * _jt + jr)
                for t in range(8):
                    idxv = ridx_v[pl.ds(128 * _k + 16 * t, 16)]
                    v = plsc.load_gather(rtab_v, [idxv, jvec])
                    rtile[jr, pl.ds(16 * t, 16)] = v
                return 0

            lax.fori_loop(0, 8, rel_row, 0)
            pltpu.sync_copy(rtile, o4.at[jt, wid * BTW + k])


def _run(entity_table, relation_table, entity_idx, relation_idx):
    mesh = plsc.VectorSubcoreMesh(core_axis_name="c", subcore_axis_name="s")
    kern = functools.partial(
        pl.kernel,
        out_type=[
            jax.ShapeDtypeStruct((L, 4, B // 128, 8, 128), jnp.float32),
            jax.ShapeDtypeStruct((4, B // 128, 8, 128), jnp.float32),
        ],
        mesh=mesh,
        compiler_params=pltpu.CompilerParams(
            use_tc_tiling_on_sc=False, needs_layout_passes=False),
        scratch_types=[
            pltpu.VMEM((BW, L), jnp.int32),
            pltpu.VMEM((NBUF, 128), jnp.int32),
            pltpu.VMEM((128, ENT_DIM), jnp.float32),
            pltpu.VMEM((128, ENT_DIM), jnp.float32),
            pltpu.VMEM((128, ENT_DIM), jnp.float32),
            pltpu.VMEM((128, ENT_DIM), jnp.float32),
            pltpu.VMEM((ENT_DIM, 128), jnp.float32),
            pltpu.VMEM((ENT_DIM, 128), jnp.float32),
            pltpu.VMEM((NREL, REL_DIM), jnp.float32),
            pltpu.VMEM((BW,), jnp.int32),
            pltpu.VMEM((8, 128), jnp.float32),
            pltpu.SemaphoreType.DMA,
            pltpu.SemaphoreType.DMA,
            pltpu.SemaphoreType.DMA,
            pltpu.SemaphoreType.DMA,
            pltpu.SemaphoreType.DMA,
            pltpu.SemaphoreType.DMA,
            pltpu.SemaphoreType.DMA,
        ],
    )(_body)
    return kern(entity_table, relation_table, entity_idx, relation_idx)


def kernel(entity_table, relation_table, entity_idx, relation_idx):
    eidx = entity_idx.astype(jnp.int32)
    ridx = relation_idx.astype(jnp.int32)
    out5, out_r4 = _run(entity_table, relation_table, eidx, ridx)
    out_e = out5.transpose(2, 4, 0, 1, 3).reshape(B, L, ENT_DIM)
    out_r = out_r4.transpose(1, 3, 0, 2).reshape(B, REL_DIM)
    return out_e, out_r


# speedup vs baseline: 1.0393x; 1.0393x over previous
import functools

import jax
import jax.numpy as jnp
from jax import lax
from jax.experimental import pallas as pl
from jax.experimental.pallas import tpu as pltpu
from jax.experimental.pallas import tpu_sc as plsc

ENT_DIM = 32
REL_DIM = 32
B = 16384
L = 50
NREL = 1000

_info = plsc.get_sparse_core_info()
NC, NS = _info.num_cores, _info.num_subcores
NW = NC * NS
BW = B // NW
BTW = BW // 128
NCHUNK = L * BTW
NBUF = 8


def _body(ent_hbm, rel_hbm, eidx_hbm, ridx_hbm, o5, o4,
          idx_v, idxc_v, gb0, gb1, gb2, gb3, gb4, gb5, gb6, gb7,
          tb0, tb1, rtab_v, ridx_v, rtile,
          sg0, sg1, sg2, sg3, sg4, sg5, sg6, sg7, sw0, sw1, srt):
    wid = lax.axis_index("s") * NC + lax.axis_index("c")
    b0 = wid * BW
    gbufs = (gb0, gb1, gb2, gb3, gb4, gb5, gb6, gb7)
    sgs = (sg0, sg1, sg2, sg3, sg4, sg5, sg6, sg7)
    tbufs = (tb0, tb1)
    sws = (sw0, sw1)
    iota = lax.iota(jnp.int32, 16)

    pltpu.async_copy(rel_hbm, rtab_v, srt)
    pltpu.sync_copy(eidx_hbm.at[pl.ds(b0, BW)], idx_v)
    pltpu.sync_copy(ridx_hbm.at[pl.ds(b0, BW)], ridx_v)

    def repack_and_fire(c, s):
        k = c // L
        l = c % L
        lvec = jnp.full((16,), 0, jnp.int32) + l
        for t in range(8):
            rows = plsc.load_gather(idx_v, [k * 128 + 16 * t + iota, lvec])
            idxc_v[s, pl.ds(16 * t, 16)] = rows
        pltpu.async_copy(ent_hbm.at[idxc_v.at[s]], gbufs[s], sgs[s])

    def wait_writes(p):
        for jt in range(4):
            pltpu.make_async_copy(tbufs[p].at[pl.ds(8 * jt, 8)],
                                  o5.at[0, jt, 0], sws[p]).wait()

    def process(c, s, p, skip_wait=False, fire_next=True):
        k = c // L
        l = c % L
        pltpu.make_async_copy(ent_hbm.at[idxc_v.at[s]], gbufs[s],
                              sgs[s]).wait()
        if fire_next is True:
            repack_and_fire(c + NBUF - 1, (s + NBUF - 1) % NBUF)
        elif fire_next == "guard":
            @pl.when(c + NBUF - 1 < NCHUNK)
            def _():
                repack_and_fire(c + NBUF - 1, (s + NBUF - 1) % NBUF)
        if not skip_wait:
            wait_writes(p)

        def tr_row(j, _):
            jvec = jnp.full((16,), 0, jnp.int32) + j
            for t in range(8):
                v = plsc.load_gather(gbufs[s], [16 * t + iota, jvec])
                tbufs[p][j, pl.ds(16 * t, 16)] = v
            return 0

        lax.fori_loop(0, ENT_DIM, tr_row, 0)
        for jt in range(4):
            pltpu.async_copy(tbufs[p].at[pl.ds(8 * jt, 8)],
                             o5.at[l, jt, wid * BTW + k], sws[p])

    for c in range(NBUF - 1):
        repack_and_fire(c, c)
    for s in range(NBUF):
        process(s, s, s % 2, skip_wait=(s < 2))

    def step(kk, _):
        for s in range(NBUF):
            c = kk * NBUF + s
            process(c, s, s % 2, fire_next="guard")
        return 0

    lax.fori_loop(1, NCHUNK // NBUF, step, 0)
    wait_writes(0)
    wait_writes(1)

    pltpu.make_async_copy(rel_hbm, rtab_v, srt).wait()
    for k in range(BTW):
        for jt in range(4):

            def rel_row(jr, _, _k=k, _jt=jt):
                jvec = jnp.full((16,), 0, jnp.int32) + (8 * _jt + jr)
                for t in range(8):
                    idxv = ridx_v[pl.ds(128 * _k + 16 * t, 16)]
                    v = plsc.load_gather(rtab_v, [idxv, jvec])
                    rtile[jr, pl.ds(16 * t, 16)] = v
                return 0

            lax.fori_loop(0, 8, rel_row, 0)
            pltpu.sync_copy(rtile, o4.at[jt, wid * BTW + k])


def _run(entity_table, relation_table, entity_idx, relation_idx):
    mesh = plsc.VectorSubcoreMesh(core_axis_name="c", subcore_axis_name="s")
    kern = functools.partial(
        pl.kernel,
        out_type=[
            jax.ShapeDtypeStruct((L, 4, B // 128, 8, 128), jnp.float32),
            jax.ShapeDtypeStruct((4, B // 128, 8, 128), jnp.float32),
        ],
        mesh=mesh,
        compiler_params=pltpu.CompilerParams(
            use_tc_tiling_on_sc=False, needs_layout_passes=False),
        scratch_types=(
            [pltpu.VMEM((BW, L), jnp.int32),
             pltpu.VMEM((NBUF, 128), jnp.int32)]
            + [pltpu.VMEM((128, ENT_DIM), jnp.float32)] * NBUF
            + [pltpu.VMEM((ENT_DIM, 128), jnp.float32)] * 2
            + [pltpu.VMEM((NREL, REL_DIM), jnp.float32),
               pltpu.VMEM((BW,), jnp.int32),
               pltpu.VMEM((8, 128), jnp.float32)]
            + [pltpu.SemaphoreType.DMA] * (NBUF + 3)
        ),
    )(_body)
    return kern(entity_table, relation_table, entity_idx, relation_idx)


def kernel(entity_table, relation_table, entity_idx, relation_idx):
    eidx = entity_idx.astype(jnp.int32)
    ridx = relation_idx.astype(jnp.int32)
    out5, out_r4 = _run(entity_table, relation_table, eidx, ridx)
    out_e = out5.transpose(2, 4, 0, 1, 3).reshape(B, L, ENT_DIM)
    out_r = out_r4.transpose(1, 3, 0, 2).reshape(B, REL_DIM)
    return out_e, out_r


# speedup vs baseline: 1.7451x; 1.6791x over previous
import functools

import jax
import jax.numpy as jnp
from jax import lax
from jax.experimental import pallas as pl
from jax.experimental.pallas import tpu as pltpu
from jax.experimental.pallas import tpu_sc as plsc

ENT_DIM = 32
REL_DIM = 32
B = 16384
L = 50
NREL = 1000
PITCH = 129

_info = plsc.get_sparse_core_info()
NC, NS = _info.num_cores, _info.num_subcores
NW = NC * NS
BW = B // NW
BTW = BW // 128
NCHUNK = L * BTW
NBUF = 4


def _body(ent_hbm, rel_hbm, eidx_hbm, ridx_hbm, o5, o4,
          idx_v, idxc_v, gb0, gb1, gb2, gb3,
          st0, st1, rtab_v, ridx_v, rtile,
          sg0, sg1, sg2, sg3, sw0, sw1, srt):
    wid = lax.axis_index("s") * NC + lax.axis_index("c")
    b0 = wid * BW
    gbufs = (gb0, gb1, gb2, gb3)
    sgs = (sg0, sg1, sg2, sg3)
    stags = (st0, st1)
    sws = (sw0, sw1)
    iota = lax.iota(jnp.int32, 16)
    iota16 = iota + 16

    pltpu.async_copy(rel_hbm, rtab_v, srt)
    pltpu.sync_copy(eidx_hbm.at[pl.ds(b0, BW)], idx_v)
    pltpu.sync_copy(ridx_hbm.at[pl.ds(b0, BW)], ridx_v)

    def repack_and_fire(c, s):
        k = c // L
        l = c % L
        lvec = jnp.full((16,), 0, jnp.int32) + l
        for t in range(8):
            rows = plsc.load_gather(idx_v, [k * 128 + 16 * t + iota, lvec])
            idxc_v[s, pl.ds(16 * t, 16)] = rows
        pltpu.async_copy(ent_hbm.at[idxc_v.at[s]], gbufs[s], sgs[s])

    def wait_writes(p):
        for j in range(ENT_DIM):
            pltpu.make_async_copy(stags[p].at[0, pl.ds(0, 128)],
                                  o5.at[0, 0, 0, 0], sws[p]).wait()

    def process(c, s, p, skip_wait=False, fire_next=True):
        k = c // L
        l = c % L
        pltpu.make_async_copy(ent_hbm.at[idxc_v.at[s]], gbufs[s],
                              sgs[s]).wait()
        if fire_next is True:
            repack_and_fire(c + NBUF - 1, (s + NBUF - 1) % NBUF)
        elif fire_next == "guard":
            @pl.when(c + NBUF - 1 < NCHUNK)
            def _():
                repack_and_fire(c + NBUF - 1, (s + NBUF - 1) % NBUF)
        if not skip_wait:
            wait_writes(p)

        def tr_b(b, _):
            bvec = jnp.full((16,), 0, jnp.int32) + b
            va = gbufs[s][b, pl.ds(0, 16)]
            vb = gbufs[s][b, pl.ds(16, 16)]
            plsc.store_scatter(stags[p], [iota, bvec], va)
            plsc.store_scatter(stags[p], [iota16, bvec], vb)
            return 0

        lax.fori_loop(0, 128, tr_b, 0)
        for j in range(ENT_DIM):
            pltpu.async_copy(stags[p].at[j, pl.ds(0, 128)],
                             o5.at[l, j // 8, wid * BTW + k, j % 8], sws[p])

    for c in range(NBUF - 1):
        repack_and_fire(c, c)
    for s in range(NBUF):
        process(s, s, s % 2, skip_wait=(s < 2))

    def step(kk, _):
        for s in range(NBUF):
            c = kk * NBUF + s
            process(c, s, s % 2, fire_next="guard")
        return 0

    lax.fori_loop(1, NCHUNK // NBUF, step, 0)
    wait_writes(0)
    wait_writes(1)

    pltpu.make_async_copy(rel_hbm, rtab_v, srt).wait()
    for k in range(BTW):
        for jt in range(4):

            def rel_row(jr, _, _k=k, _jt=jt):
                jvec = jnp.full((16,), 0, jnp.int32) + (8 * _jt + jr)
                for t in range(8):
                    idxv = ridx_v[pl.ds(128 * _k + 16 * t, 16)]
                    v = plsc.load_gather(rtab_v, [idxv, jvec])
                    rtile[jr, pl.ds(16 * t, 16)] = v
                return 0

            lax.fori_loop(0, 8, rel_row, 0)
            pltpu.sync_copy(rtile, o4.at[jt, wid * BTW + k])


def _run(entity_table, relation_table, entity_idx, relation_idx):
    mesh = plsc.VectorSubcoreMesh(core_axis_name="c", subcore_axis_name="s")
    kern = functools.partial(
        pl.kernel,
        out_type=[
            jax.ShapeDtypeStruct((L, 4, B // 128, 8, 128), jnp.float32),
            jax.ShapeDtypeStruct((4, B // 128, 8, 128), jnp.float32),
        ],
        mesh=mesh,
        compiler_params=pltpu.CompilerParams(
            use_tc_tiling_on_sc=False, needs_layout_passes=False),
        scratch_types=(
            [pltpu.VMEM((BW, L), jnp.int32),
             pltpu.VMEM((NBUF, 128), jnp.int32)]
            + [pltpu.VMEM((128, ENT_DIM), jnp.float32)] * NBUF
            + [pltpu.VMEM((ENT_DIM, PITCH), jnp.float32)] * 2
            + [pltpu.VMEM((NREL, REL_DIM), jnp.float32),
               pltpu.VMEM((BW,), jnp.int32),
               pltpu.VMEM((8, 128), jnp.float32)]
            + [pltpu.SemaphoreType.DMA] * (NBUF + 3)
        ),
    )(_body)
    return kern(entity_table, relation_table, entity_idx, relation_idx)


def kernel(entity_table, relation_table, entity_idx, relation_idx):
    eidx = entity_idx.astype(jnp.int32)
    ridx = relation_idx.astype(jnp.int32)
    out5, out_r4 = _run(entity_table, relation_table, eidx, ridx)
    out_e = out5.transpose(2, 4, 0, 1, 3).reshape(B, L, ENT_DIM)
    out_r = out_r4.transpose(1, 3, 0, 2).reshape(B, REL_DIM)
    return out_e, out_r


# speedup vs baseline: 1.7599x; 1.0085x over previous
import functools

import jax
import jax.numpy as jnp
from jax import lax
from jax.experimental import pallas as pl
from jax.experimental.pallas import tpu as pltpu
from jax.experimental.pallas import tpu_sc as plsc

ENT_DIM = 32
REL_DIM = 32
B = 16384
L = 50
NREL = 1000
PITCH = 129

_info = plsc.get_sparse_core_info()
NC, NS = _info.num_cores, _info.num_subcores
NW = NC * NS
BW = B // NW
BTW = BW // 128
NCHUNK = L * BTW
NBUF = 4


def _body(ent_hbm, rel_hbm, eidx_hbm, ridx_hbm, o5, o4,
          idx_v, idxc_v, gb0, gb1, gb2, gb3,
          st0, st1, rtab_v, ridx_v, rtile,
          sg0, sg1, sg2, sg3, sw0, sw1, srt):
    wid = lax.axis_index("s") * NC + lax.axis_index("c")
    b0 = wid * BW
    gbufs = (gb0, gb1, gb2, gb3)
    sgs = (sg0, sg1, sg2, sg3)
    stags = (st0, st1)
    sws = (sw0, sw1)
    iota = lax.iota(jnp.int32, 16)
    iota16 = iota + 16

    pltpu.async_copy(rel_hbm, rtab_v, srt)
    pltpu.sync_copy(eidx_hbm.at[pl.ds(b0, BW)], idx_v)
    pltpu.sync_copy(ridx_hbm.at[pl.ds(b0, BW)], ridx_v)

    def repack_and_fire(c, s):
        k = c // L
        l = c % L
        lvec = jnp.full((16,), 0, jnp.int32) + l
        for t in range(8):
            rows = plsc.load_gather(idx_v, [k * 128 + 16 * t + iota, lvec])
            idxc_v[s, pl.ds(16 * t, 16)] = rows
        pltpu.async_copy(ent_hbm.at[idxc_v.at[s]], gbufs[s], sgs[s])

    def wait_writes(p):
        for j in range(ENT_DIM):
            pltpu.make_async_copy(stags[p].at[0, pl.ds(0, 128)],
                                  o5.at[0, 0, 0, 0], sws[p]).wait()

    def process(c, s, p, skip_wait=False, fire_next=True):
        k = c // L
        l = c % L
        pltpu.make_async_copy(ent_hbm.at[idxc_v.at[s]], gbufs[s],
                              sgs[s]).wait()
        if fire_next is True:
            repack_and_fire(c + NBUF - 1, (s + NBUF - 1) % NBUF)
        elif fire_next == "guard":
            @pl.when(c + NBUF - 1 < NCHUNK)
            def _():
                repack_and_fire(c + NBUF - 1, (s + NBUF - 1) % NBUF)
        if not skip_wait:
            wait_writes(p)

        def tr_b(b4, _):
            b = b4 * 4
            for u in range(4):
                bvec = jnp.full((16,), u, jnp.int32) + b
                va = gbufs[s][b + u, pl.ds(0, 16)]
                vb = gbufs[s][b + u, pl.ds(16, 16)]
                plsc.store_scatter(stags[p], [iota, bvec], va)
                plsc.store_scatter(stags[p], [iota16, bvec], vb)
            return 0

        lax.fori_loop(0, 32, tr_b, 0)
        for j in range(ENT_DIM):
            pltpu.async_copy(stags[p].at[j, pl.ds(0, 128)],
                             o5.at[l, j // 8, wid * BTW + k, j % 8], sws[p])

    for c in range(NBUF - 1):
        repack_and_fire(c, c)
    for s in range(NBUF):
        process(s, s, s % 2, skip_wait=(s < 2))

    def step(kk, _):
        for s in range(NBUF):
            c = kk * NBUF + s
            process(c, s, s % 2, fire_next="guard")
        return 0

    lax.fori_loop(1, NCHUNK // NBUF, step, 0)
    wait_writes(0)
    wait_writes(1)

    pltpu.make_async_copy(rel_hbm, rtab_v, srt).wait()
    for k in range(BTW):
        for jt in range(4):

            def rel_row(jr, _, _k=k, _jt=jt):
                jvec = jnp.full((16,), 0, jnp.int32) + (8 * _jt + jr)
                for t in range(8):
                    idxv = ridx_v[pl.ds(128 * _k + 16 * t, 16)]
                    v = plsc.load_gather(rtab_v, [idxv, jvec])
                    rtile[jr, pl.ds(16 * t, 16)] = v
                return 0

            lax.fori_loop(0, 8, rel_row, 0)
            pltpu.sync_copy(rtile, o4.at[jt, wid * BTW + k])


def _run(entity_table, relation_table, entity_idx, relation_idx):
    mesh = plsc.VectorSubcoreMesh(core_axis_name="c", subcore_axis_name="s")
    kern = functools.partial(
        pl.kernel,
        out_type=[
            jax.ShapeDtypeStruct((L, 4, B // 128, 8, 128), jnp.float32),
            jax.ShapeDtypeStruct((4, B // 128, 8, 128), jnp.float32),
        ],
        mesh=mesh,
        compiler_params=pltpu.CompilerParams(
            use_tc_tiling_on_sc=False, needs_layout_passes=False),
        scratch_types=(
            [pltpu.VMEM((BW, L), jnp.int32),
             pltpu.VMEM((NBUF, 128), jnp.int32)]
            + [pltpu.VMEM((128, ENT_DIM), jnp.float32)] * NBUF
            + [pltpu.VMEM((ENT_DIM, PITCH), jnp.float32)] * 2
            + [pltpu.VMEM((NREL, REL_DIM), jnp.float32),
               pltpu.VMEM((BW,), jnp.int32),
               pltpu.VMEM((8, 128), jnp.float32)]
            + [pltpu.SemaphoreType.DMA] * (NBUF + 3)
        ),
    )(_body)
    return kern(entity_table, relation_table, entity_idx, relation_idx)


def kernel(entity_table, relation_table, entity_idx, relation_idx):
    eidx = entity_idx.astype(jnp.int32)
    ridx = relation_idx.astype(jnp.int32)
    out5, out_r4 = _run(entity_table, relation_table, eidx, ridx)
    out_e = out5.transpose(2, 4, 0, 1, 3).reshape(B, L, ENT_DIM)
    out_r = out_r4.transpose(1, 3, 0, 2).reshape(B, REL_DIM)
    return out_e, out_r
